# baseline (speedup 1.0000x reference)
import functools

import jax
import jax.numpy as jnp
from jax import lax
from jax.experimental import pallas as pl
from jax.experimental.pallas import tpu as pltpu
from jax.experimental.pallas import tpu_sc as plsc

_N = 10000
_E = 320000
_G = 64

_NC = 2
_NS = 16
_NW = _NC * _NS

_EPW = _E // _NW
_EPS = _E // _NS
_CH = 128
_NFULL = _EPW // _CH
_TAIL = _EPW - _NFULL * _CH

_NPAD = 10240
_RPT = _NPAD // _NS
_SPT = _N // _NS
_BLK = 2000
_GRID = _N // _BLK


def _wid(cid, sid):
    return cid * _NS + sid


def _deg_body(ei16, out, outw, dst_v, ones_v, wbuf_v, wide_v, deg_sh):
    cid = lax.axis_index("c")
    sid = lax.axis_index("s")

    ones16 = jnp.full((16,), 1.0, jnp.float32)

    def ob(i, _):
        ones_v[pl.ds(i * 16, 16)] = ones16
        return ()
    lax.fori_loop(0, _EPW // 16, ob, ())
    zeros16 = jnp.zeros((16,), jnp.float32)

    def zb(i, _):
        wbuf_v[pl.ds(i * 16, 16)] = zeros16
        return ()
    lax.fori_loop(0, _RPT // 16, zb, ())
    pltpu.sync_copy(wbuf_v, deg_sh.at[pl.ds(sid * _RPT, _RPT)])
    plsc.subcore_barrier()

    pltpu.sync_copy(ei16.at[1, sid], dst_v)
    pltpu.sync_copy(ones_v, deg_sh.at[dst_v.at[pl.ds(0, _EPW)]], add=True)
    pltpu.sync_copy(ones_v, deg_sh.at[dst_v.at[pl.ds(_EPW, _EPW)]], add=True)
    plsc.subcore_barrier()

    pltpu.sync_copy(deg_sh.at[pl.ds(sid * _RPT, _RPT)], wbuf_v)
    magic = jnp.full((16,), 0x5F3759DF, jnp.int32)

    def nw(i, _):
        x = wbuf_v[pl.ds(i * 16, 16)] + 1.0
        y = plsc.bitcast(magic - (plsc.bitcast(x, jnp.int32) >> 1),
                         jnp.float32)
        for _u in range(3):
            y = y * (1.5 - 0.5 * x * y * y)
        wbuf_v[pl.ds(i * 16, 16)] = y
        for k in range(16):
            r = i * 16 + k
            dvv = jnp.broadcast_to(y[k], (16,))
            dvb = plsc.pack(dvv, dvv, format=plsc.PackFormat.INTERLEAVED)
            for j in range(2):
                wide_v[pl.ds(r * 64 + j * 32, 32)] = dvb
        return ()
    lax.fori_loop(0, _RPT // 16, nw, ())

    @pl.when(cid == 0)
    def _():
        pltpu.sync_copy(wbuf_v, out.at[pl.ds(sid * _RPT, _RPT)])
        pltpu.sync_copy(wide_v, outw.at[pl.ds(sid * _RPT * 64, _RPT * 64)])


def _deg_sc(ei16):
    mesh = plsc.VectorSubcoreMesh(core_axis_name="c", subcore_axis_name="s")
    return pl.kernel(
        _deg_body,
        out_type=(jax.ShapeDtypeStruct((_NPAD,), jnp.float32),
                  jax.ShapeDtypeStruct((_NPAD * 64,), jnp.bfloat16)),
        mesh=mesh,
        compiler_params=pltpu.CompilerParams(use_tc_tiling_on_sc=False,
                                             needs_layout_passes=False),
        scratch_types=[
            pltpu.VMEM((_EPS,), jnp.int32),
            pltpu.VMEM((_EPW,), jnp.float32),
            pltpu.VMEM((_RPT,), jnp.float32),
            pltpu.VMEM((_RPT * 64,), jnp.bfloat16),
            pltpu.VMEM_SHARED((_NPAD,), jnp.float32),
        ],
    )(ei16)


def _edge_pass(g_sh, acc_sh, src_v, dst_v, rows_v):
    def step(j, _):
        pltpu.sync_copy(g_sh.at[src_v.at[pl.ds(j * _CH, _CH)]], rows_v)
        pltpu.sync_copy(rows_v, acc_sh.at[dst_v.at[pl.ds(j * _CH, _CH)]],
                        add=True)
        return ()
    lax.fori_loop(0, _NFULL, step, ())
    pltpu.sync_copy(g_sh.at[src_v.at[pl.ds(_NFULL * _CH, _TAIL)]],
                    rows_v.at[pl.ds(0, _TAIL)])
    pltpu.sync_copy(rows_v.at[pl.ds(0, _TAIL)],
                    acc_sh.at[dst_v.at[pl.ds(_NFULL * _CH, _TAIL)]],
                    add=True)


def _zero_acc(zbuf_v, acc_sh, sid):
    zeros32 = jnp.zeros((32,), jnp.bfloat16)

    def zb(i, _):
        for j in range(2):
            zbuf_v[i, pl.ds(j * 32, 32)] = zeros32
        return ()
    lax.fori_loop(0, 64, zb, ())
    for r in range(_RPT // 64):
        pltpu.sync_copy(zbuf_v, acc_sh.at[pl.ds(sid * _RPT + r * 64, 64)])


def _conv1_body(h, ei3, dinvw, out, src_v, dst_v, rows_v, zbuf_v, hbuf_v,
                dwbuf_v, acc_sh, g_sh):
    cid = lax.axis_index("c")
    sid = lax.axis_index("s")

    last = _NS - 1
    ltr = _N - last * _RPT

    @pl.when(sid < last)
    def _():
        pltpu.sync_copy(h.at[pl.ds(sid * _RPT, _RPT)], hbuf_v)
        pltpu.sync_copy(dinvw.at[pl.ds(sid * _RPT, _RPT)], dwbuf_v)

    @pl.when(sid == last)
    def _():
        pltpu.sync_copy(h.at[pl.ds(last * _RPT, ltr)],
                        hbuf_v.at[pl.ds(0, ltr)])
        pltpu.sync_copy(dinvw.at[pl.ds(last * _RPT, ltr)],
                        dwbuf_v.at[pl.ds(0, ltr)])
    _zero_acc(zbuf_v, acc_sh, sid)

    def sc_row(r, _):
        for j in range(2):
            hbuf_v[r, pl.ds(j * 32, 32)] = \
                hbuf_v[r, pl.ds(j * 32, 32)] * dwbuf_v[r, pl.ds(j * 32, 32)]
        return ()
    nrow = jnp.where(sid == last, ltr, _RPT)
    lax.fori_loop(0, nrow, sc_row, ())

    @pl.when(sid < last)
    def _():
        pltpu.sync_copy(hbuf_v, g_sh.at[pl.ds(sid * _RPT, _RPT)])

    @pl.when(sid == last)
    def _():
        pltpu.sync_copy(hbuf_v.at[pl.ds(0, ltr)],
                        g_sh.at[pl.ds(last * _RPT, ltr)])
    plsc.subcore_barrier()

    w = _wid(cid, sid)
    pltpu.sync_copy(ei3.at[0, w], src_v)
    pltpu.sync_copy(ei3.at[1, w], dst_v)
    _edge_pass(g_sh, acc_sh, src_v, dst_v, rows_v)
    plsc.subcore_barrier()

    pltpu.sync_copy(acc_sh.at[pl.ds(sid * _RPT, _RPT)],
                    out.at[cid, pl.ds(sid * _RPT, _RPT)])


def _conv2_body(g, ei3, out, src_v, dst_v, rows_v, zbuf_v, acc_sh, g_sh):
    cid = lax.axis_index("c")
    sid = lax.axis_index("s")

    pltpu.sync_copy(g.at[pl.ds(sid * _SPT, _SPT)],
                    g_sh.at[pl.ds(sid * _SPT, _SPT)])
    _zero_acc(zbuf_v, acc_sh, sid)
    plsc.subcore_barrier()

    w = _wid(cid, sid)
    pltpu.sync_copy(ei3.at[0, w], src_v)
    pltpu.sync_copy(ei3.at[1, w], dst_v)
    _edge_pass(g_sh, acc_sh, src_v, dst_v, rows_v)
    plsc.subcore_barrier()

    pltpu.sync_copy(acc_sh.at[pl.ds(sid * _RPT, _RPT)],
                    out.at[cid, pl.ds(sid * _RPT, _RPT)])


_CONV_OUT = jax.ShapeDtypeStruct((_NC, _NPAD, 64), jnp.bfloat16)
_CONV_SCRATCH = [
    pltpu.VMEM((_EPW,), jnp.int32),
    pltpu.VMEM((_EPW,), jnp.int32),
    pltpu.VMEM((_CH, 64), jnp.bfloat16),
    pltpu.VMEM((64, 64), jnp.bfloat16),
]
_CONV_SHARED = [
    pltpu.VMEM_SHARED((_NPAD, 64), jnp.bfloat16),
    pltpu.VMEM_SHARED((_NPAD, 64), jnp.bfloat16),
]


def _conv1_sc(h, ei3, dinvw):
    mesh = plsc.VectorSubcoreMesh(core_axis_name="c", subcore_axis_name="s")
    return pl.kernel(
        _conv1_body,
        out_type=_CONV_OUT,
        mesh=mesh,
        compiler_params=pltpu.CompilerParams(use_tc_tiling_on_sc=False),
        scratch_types=_CONV_SCRATCH + [
            pltpu.VMEM((_RPT, 64), jnp.bfloat16),
            pltpu.VMEM((_RPT, 64), jnp.bfloat16),
        ] + _CONV_SHARED,
    )(h, ei3, dinvw)


def _conv2_sc(g, ei3):
    mesh = plsc.VectorSubcoreMesh(core_axis_name="c", subcore_axis_name="s")
    return pl.kernel(
        _conv2_body,
        out_type=_CONV_OUT,
        mesh=mesh,
        compiler_params=pltpu.CompilerParams(use_tc_tiling_on_sc=False),
        scratch_types=_CONV_SCRATCH + _CONV_SHARED,
    )(g, ei3)


def _mm1_body(x_ref, w_ref, h_ref):
    h_ref[...] = jnp.dot(x_ref[...], w_ref[...],
                         preferred_element_type=jnp.float32
                         ).astype(jnp.bfloat16)


def _mid_body(ap_ref, h_ref, dv_ref, b1_ref, w2_ref, g2_ref):
    dinv = dv_ref[...]
    g1 = dinv * h_ref[...].astype(jnp.float32)
    acc = (ap_ref[0] + ap_ref[1]).astype(jnp.float32) + g1
    h1 = jnp.maximum(dinv * acc + b1_ref[...], 0.0)
    g2_ref[...] = (dinv * jnp.dot(h1, w2_ref[...],
                                  preferred_element_type=jnp.float32)
                   ).astype(jnp.bfloat16)


def _pool_body(ap_ref, g2_ref, dv_ref, b2_ref, bat_ref, out_ref):
    i = pl.program_id(0)
    dinv = dv_ref[...]
    h2 = dinv * ((ap_ref[0] + ap_ref[1]).astype(jnp.float32)
                 + g2_ref[...].astype(jnp.float32)) + b2_ref[...]
    ids = jax.lax.broadcasted_iota(jnp.int32, (_G, _BLK), 0)
    oht = (ids == bat_ref[0]).astype(jnp.float32)
    part = jnp.dot(oht, h2, preferred_element_type=jnp.float32)

    @pl.when(i == 0)
    def _():
        out_ref[...] = part

    @pl.when(i > 0)
    def _():
        out_ref[...] += part


def _mm1_tc(x, W1):
    return pl.pallas_call(
        _mm1_body,
        grid=(_GRID,),
        in_specs=[pl.BlockSpec((_BLK, 128), lambda i: (i, 0)),
                  pl.BlockSpec((128, 64), lambda i: (0, 0))],
        out_specs=pl.BlockSpec((_BLK, 64), lambda i: (i, 0)),
        out_shape=jax.ShapeDtypeStruct((_N, 64), jnp.bfloat16),
    )(x, W1)


def _mid_tc(ap, h, dinvc, b1, W2):
    return pl.pallas_call(
        _mid_body,
        grid=(_GRID,),
        in_specs=[pl.BlockSpec((_NC, _BLK, 64), lambda i: (0, i, 0)),
                  pl.BlockSpec((_BLK, 64), lambda i: (i, 0)),
                  pl.BlockSpec((_BLK, 1), lambda i: (i, 0)),
                  pl.BlockSpec((1, 64), lambda i: (0, 0)),
                  pl.BlockSpec((64, 64), lambda i: (0, 0))],
        out_specs=pl.BlockSpec((_BLK, 64), lambda i: (i, 0)),
        out_shape=jax.ShapeDtypeStruct((_N, 64), jnp.bfloat16),
    )(ap, h, dinvc, b1, W2)


def _pool_tc(ap, g2, dinvc, b2, bat3):
    return pl.pallas_call(
        _pool_body,
        grid=(_GRID,),
        in_specs=[pl.BlockSpec((_NC, _BLK, 64), lambda i: (0, i, 0)),
                  pl.BlockSpec((_BLK, 64), lambda i: (i, 0)),
                  pl.BlockSpec((_BLK, 1), lambda i: (i, 0)),
                  pl.BlockSpec((1, 64), lambda i: (0, 0)),
                  pl.BlockSpec((1, 1, _BLK), lambda i: (i, 0, 0))],
        out_specs=pl.BlockSpec((_G, 64), lambda i: (0, 0)),
        out_shape=jax.ShapeDtypeStruct((_G, 64), jnp.float32),
    )(ap, g2, dinvc, b2, bat3)


def kernel(x, edge_index, batch, W1, b1, W2, b2):
    ei3 = edge_index.reshape(2, _NW, _EPW)
    ei16 = edge_index.reshape(2, _NS, _EPS)
    bat3 = batch.reshape(_GRID, 1, _BLK)

    dinv, dinvw = _deg_sc(ei16)
    h = _mm1_tc(x, W1)
    dinvc = dinv.reshape(_NPAD, 1)

    ap1 = _conv1_sc(h, ei3, dinvw.reshape(_NPAD, 64))
    g2 = _mid_tc(ap1, h, dinvc, b1.reshape(1, 64), W2)
    ap2 = _conv2_sc(g2, ei3)
    out = _pool_tc(ap2, g2, dinvc, b2.reshape(1, 64), bat3)
    return out


# speedup vs baseline: 1.1497x; 1.1497x over previous
import functools

import jax
import jax.numpy as jnp
from jax import lax
from jax.experimental import pallas as pl
from jax.experimental.pallas import tpu as pltpu
from jax.experimental.pallas import tpu_sc as plsc

_N = 10000
_E = 320000
_G = 64

_NC = 2
_NS = 16
_NW = _NC * _NS

_EPW = _E // _NW
_EPS = _E // _NS
_CH = 128
_NFULL = _EPW // _CH
_TAIL = _EPW - _NFULL * _CH

_NPAD = 10240
_RPT = _NPAD // _NS
_SPT = _N // _NS
_BLK = 2000
_GRID = _N // _BLK


def _wid(cid, sid):
    return cid * _NS + sid


def _deg_body(ei16, out, outw, dst_v, ones_v, wbuf_v, wide_v, deg_sh):
    cid = lax.axis_index("c")
    sid = lax.axis_index("s")

    ones16 = jnp.full((16,), 1.0, jnp.float32)

    def ob(i, _):
        ones_v[pl.ds(i * 16, 16)] = ones16
        return ()
    lax.fori_loop(0, _EPW // 16, ob, ())
    zeros16 = jnp.zeros((16,), jnp.float32)

    def zb(i, _):
        wbuf_v[pl.ds(i * 16, 16)] = zeros16
        return ()
    lax.fori_loop(0, _RPT // 16, zb, ())
    pltpu.sync_copy(wbuf_v, deg_sh.at[pl.ds(sid * _RPT, _RPT)])
    plsc.subcore_barrier()

    pltpu.sync_copy(ei16.at[1, sid], dst_v)
    pltpu.sync_copy(ones_v, deg_sh.at[dst_v.at[pl.ds(0, _EPW)]], add=True)
    pltpu.sync_copy(ones_v, deg_sh.at[dst_v.at[pl.ds(_EPW, _EPW)]], add=True)
    plsc.subcore_barrier()

    pltpu.sync_copy(deg_sh.at[pl.ds(sid * _RPT, _RPT)], wbuf_v)
    magic = jnp.full((16,), 0x5F3759DF, jnp.int32)

    def nw(i, _):
        x = wbuf_v[pl.ds(i * 16, 16)] + 1.0
        y = plsc.bitcast(magic - (plsc.bitcast(x, jnp.int32) >> 1),
                         jnp.float32)
        for _u in range(3):
            y = y * (1.5 - 0.5 * x * y * y)
        wbuf_v[pl.ds(i * 16, 16)] = y
        for k in range(16):
            r = i * 16 + k
            dvv = jnp.broadcast_to(y[k], (16,))
            dvb = plsc.pack(dvv, dvv, format=plsc.PackFormat.INTERLEAVED)
            for j in range(2):
                wide_v[pl.ds(r * 64 + j * 32, 32)] = dvb
        return ()
    lax.fori_loop(0, _RPT // 16, nw, ())

    @pl.when(cid == 0)
    def _():
        pltpu.sync_copy(wbuf_v, out.at[pl.ds(sid * _RPT, _RPT)])
        pltpu.sync_copy(wide_v, outw.at[pl.ds(sid * _RPT * 64, _RPT * 64)])


def _deg_sc(ei16):
    mesh = plsc.VectorSubcoreMesh(core_axis_name="c", subcore_axis_name="s")
    return pl.kernel(
        _deg_body,
        out_type=(jax.ShapeDtypeStruct((_NPAD,), jnp.float32),
                  jax.ShapeDtypeStruct((_NPAD * 64,), jnp.bfloat16)),
        mesh=mesh,
        compiler_params=pltpu.CompilerParams(use_tc_tiling_on_sc=False,
                                             needs_layout_passes=False),
        scratch_types=[
            pltpu.VMEM((_EPS,), jnp.int32),
            pltpu.VMEM((_EPW,), jnp.float32),
            pltpu.VMEM((_RPT,), jnp.float32),
            pltpu.VMEM((_RPT * 64,), jnp.bfloat16),
            pltpu.VMEM_SHARED((_NPAD,), jnp.float32),
        ],
    )(ei16)


def _edge_pass(g_sh, acc_sh, src_v, dst_v, rows_v, rowsb_v, gs0, gs1,
               ss0, ss1):
    bufs = (rows_v, rowsb_v)
    gsem = (gs0, gs1)
    ssem = (ss0, ss1)

    def sg(j, b):
        return pltpu.async_copy(g_sh.at[src_v.at[pl.ds(j * _CH, _CH)]],
                                bufs[b], gsem[b])

    def ss(j, b):
        return pltpu.async_copy(bufs[b],
                                acc_sh.at[dst_v.at[pl.ds(j * _CH, _CH)]],
                                ssem[b], add=True)

    gh = [sg(0, 0), None]
    sh = [None, None]
    for j in range(_NFULL):
        b = j & 1
        if j + 1 < _NFULL:
            if sh[1 - b] is not None:
                sh[1 - b].wait()
            gh[1 - b] = sg(j + 1, 1 - b)
        gh[b].wait()
        sh[b] = ss(j, b)
    sh[(_NFULL - 2) & 1].wait()
    sh[(_NFULL - 1) & 1].wait()
    pltpu.sync_copy(g_sh.at[src_v.at[pl.ds(_NFULL * _CH, _TAIL)]],
                    rows_v.at[pl.ds(0, _TAIL)])
    pltpu.sync_copy(rows_v.at[pl.ds(0, _TAIL)],
                    acc_sh.at[dst_v.at[pl.ds(_NFULL * _CH, _TAIL)]],
                    add=True)


def _zero_acc(zbuf_v, acc_sh, sid):
    zeros32 = jnp.zeros((32,), jnp.bfloat16)

    def zb(i, _):
        for j in range(2):
            zbuf_v[i, pl.ds(j * 32, 32)] = zeros32
        return ()
    lax.fori_loop(0, 64, zb, ())
    for r in range(_RPT // 64):
        pltpu.sync_copy(zbuf_v, acc_sh.at[pl.ds(sid * _RPT + r * 64, 64)])


def _conv1_body(h, ei3, dinvw, out, src_v, dst_v, rows_v, rowsb_v, zbuf_v,
                gs0, gs1, ss0, ss1, hbuf_v, dwbuf_v, acc_sh, g_sh):
    cid = lax.axis_index("c")
    sid = lax.axis_index("s")

    last = _NS - 1
    ltr = _N - last * _RPT

    @pl.when(sid < last)
    def _():
        pltpu.sync_copy(h.at[pl.ds(sid * _RPT, _RPT)], hbuf_v)
        pltpu.sync_copy(dinvw.at[pl.ds(sid * _RPT, _RPT)], dwbuf_v)

    @pl.when(sid == last)
    def _():
        pltpu.sync_copy(h.at[pl.ds(last * _RPT, ltr)],
                        hbuf_v.at[pl.ds(0, ltr)])
        pltpu.sync_copy(dinvw.at[pl.ds(last * _RPT, ltr)],
                        dwbuf_v.at[pl.ds(0, ltr)])
    _zero_acc(zbuf_v, acc_sh, sid)

    def sc_row(r, _):
        for j in range(2):
            hbuf_v[r, pl.ds(j * 32, 32)] = \
                hbuf_v[r, pl.ds(j * 32, 32)] * dwbuf_v[r, pl.ds(j * 32, 32)]
        return ()
    nrow = jnp.where(sid == last, ltr, _RPT)
    lax.fori_loop(0, nrow, sc_row, ())

    @pl.when(sid < last)
    def _():
        pltpu.sync_copy(hbuf_v, g_sh.at[pl.ds(sid * _RPT, _RPT)])

    @pl.when(sid == last)
    def _():
        pltpu.sync_copy(hbuf_v.at[pl.ds(0, ltr)],
                        g_sh.at[pl.ds(last * _RPT, ltr)])
    plsc.subcore_barrier()

    w = _wid(cid, sid)
    pltpu.sync_copy(ei3.at[0, w], src_v)
    pltpu.sync_copy(ei3.at[1, w], dst_v)
    _edge_pass(g_sh, acc_sh, src_v, dst_v, rows_v, rowsb_v,
               gs0, gs1, ss0, ss1)
    plsc.subcore_barrier()

    pltpu.sync_copy(acc_sh.at[pl.ds(sid * _RPT, _RPT)],
                    out.at[cid, pl.ds(sid * _RPT, _RPT)])


def _conv2_body(g, ei3, out, src_v, dst_v, rows_v, rowsb_v, zbuf_v,
                gs0, gs1, ss0, ss1, acc_sh, g_sh):
    cid = lax.axis_index("c")
    sid = lax.axis_index("s")

    pltpu.sync_copy(g.at[pl.ds(sid * _SPT, _SPT)],
                    g_sh.at[pl.ds(sid * _SPT, _SPT)])
    _zero_acc(zbuf_v, acc_sh, sid)
    plsc.subcore_barrier()

    w = _wid(cid, sid)
    pltpu.sync_copy(ei3.at[0, w], src_v)
    pltpu.sync_copy(ei3.at[1, w], dst_v)
    _edge_pass(g_sh, acc_sh, src_v, dst_v, rows_v, rowsb_v,
               gs0, gs1, ss0, ss1)
    plsc.subcore_barrier()

    pltpu.sync_copy(acc_sh.at[pl.ds(sid * _RPT, _RPT)],
                    out.at[cid, pl.ds(sid * _RPT, _RPT)])


_CONV_OUT = jax.ShapeDtypeStruct((_NC, _NPAD, 64), jnp.bfloat16)
_CONV_SCRATCH = [
    pltpu.VMEM((_EPW,), jnp.int32),
    pltpu.VMEM((_EPW,), jnp.int32),
    pltpu.VMEM((_CH, 64), jnp.bfloat16),
    pltpu.VMEM((_CH, 64), jnp.bfloat16),
    pltpu.VMEM((64, 64), jnp.bfloat16),
    pltpu.SemaphoreType.DMA,
    pltpu.SemaphoreType.DMA,
    pltpu.SemaphoreType.DMA,
    pltpu.SemaphoreType.DMA,
]
_CONV_SHARED = [
    pltpu.VMEM_SHARED((_NPAD, 64), jnp.bfloat16),
    pltpu.VMEM_SHARED((_NPAD, 64), jnp.bfloat16),
]


def _conv1_sc(h, ei3, dinvw):
    mesh = plsc.VectorSubcoreMesh(core_axis_name="c", subcore_axis_name="s")
    return pl.kernel(
        _conv1_body,
        out_type=_CONV_OUT,
        mesh=mesh,
        compiler_params=pltpu.CompilerParams(use_tc_tiling_on_sc=False),
        scratch_types=_CONV_SCRATCH + [
            pltpu.VMEM((_RPT, 64), jnp.bfloat16),
            pltpu.VMEM((_RPT, 64), jnp.bfloat16),
        ] + _CONV_SHARED,
    )(h, ei3, dinvw)


def _conv2_sc(g, ei3):
    mesh = plsc.VectorSubcoreMesh(core_axis_name="c", subcore_axis_name="s")
    return pl.kernel(
        _conv2_body,
        out_type=_CONV_OUT,
        mesh=mesh,
        compiler_params=pltpu.CompilerParams(use_tc_tiling_on_sc=False),
        scratch_types=_CONV_SCRATCH + _CONV_SHARED,
    )(g, ei3)


def _mm1_body(x_ref, w_ref, h_ref):
    h_ref[...] = jnp.dot(x_ref[...], w_ref[...],
                         preferred_element_type=jnp.float32
                         ).astype(jnp.bfloat16)


def _mid_body(ap_ref, h_ref, dv_ref, b1_ref, w2_ref, g2_ref):
    dinv = dv_ref[...]
    g1 = dinv * h_ref[...].astype(jnp.float32)
    acc = (ap_ref[0] + ap_ref[1]).astype(jnp.float32) + g1
    h1 = jnp.maximum(dinv * acc + b1_ref[...], 0.0)
    g2_ref[...] = (dinv * jnp.dot(h1, w2_ref[...],
                                  preferred_element_type=jnp.float32)
                   ).astype(jnp.bfloat16)


def _pool_body(ap_ref, g2_ref, dv_ref, b2_ref, bat_ref, out_ref):
    i = pl.program_id(0)
    dinv = dv_ref[...]
    h2 = dinv * ((ap_ref[0] + ap_ref[1]).astype(jnp.float32)
                 + g2_ref[...].astype(jnp.float32)) + b2_ref[...]
    ids = jax.lax.broadcasted_iota(jnp.int32, (_G, _BLK), 0)
    oht = (ids == bat_ref[0]).astype(jnp.float32)
    part = jnp.dot(oht, h2, preferred_element_type=jnp.float32)

    @pl.when(i == 0)
    def _():
        out_ref[...] = part

    @pl.when(i > 0)
    def _():
        out_ref[...] += part


def _mm1_tc(x, W1):
    return pl.pallas_call(
        _mm1_body,
        grid=(_GRID,),
        in_specs=[pl.BlockSpec((_BLK, 128), lambda i: (i, 0)),
                  pl.BlockSpec((128, 64), lambda i: (0, 0))],
        out_specs=pl.BlockSpec((_BLK, 64), lambda i: (i, 0)),
        out_shape=jax.ShapeDtypeStruct((_N, 64), jnp.bfloat16),
    )(x, W1)


def _mid_tc(ap, h, dinvc, b1, W2):
    return pl.pallas_call(
        _mid_body,
        grid=(_GRID,),
        in_specs=[pl.BlockSpec((_NC, _BLK, 64), lambda i: (0, i, 0)),
                  pl.BlockSpec((_BLK, 64), lambda i: (i, 0)),
                  pl.BlockSpec((_BLK, 1), lambda i: (i, 0)),
                  pl.BlockSpec((1, 64), lambda i: (0, 0)),
                  pl.BlockSpec((64, 64), lambda i: (0, 0))],
        out_specs=pl.BlockSpec((_BLK, 64), lambda i: (i, 0)),
        out_shape=jax.ShapeDtypeStruct((_N, 64), jnp.bfloat16),
    )(ap, h, dinvc, b1, W2)


def _pool_tc(ap, g2, dinvc, b2, bat3):
    return pl.pallas_call(
        _pool_body,
        grid=(_GRID,),
        in_specs=[pl.BlockSpec((_NC, _BLK, 64), lambda i: (0, i, 0)),
                  pl.BlockSpec((_BLK, 64), lambda i: (i, 0)),
                  pl.BlockSpec((_BLK, 1), lambda i: (i, 0)),
                  pl.BlockSpec((1, 64), lambda i: (0, 0)),
                  pl.BlockSpec((1, 1, _BLK), lambda i: (i, 0, 0))],
        out_specs=pl.BlockSpec((_G, 64), lambda i: (0, 0)),
        out_shape=jax.ShapeDtypeStruct((_G, 64), jnp.float32),
    )(ap, g2, dinvc, b2, bat3)


def kernel(x, edge_index, batch, W1, b1, W2, b2):
    ei3 = edge_index.reshape(2, _NW, _EPW)
    ei16 = edge_index.reshape(2, _NS, _EPS)
    bat3 = batch.reshape(_GRID, 1, _BLK)

    dinv, dinvw = _deg_sc(ei16)
    h = _mm1_tc(x, W1)
    dinvc = dinv.reshape(_NPAD, 1)

    ap1 = _conv1_sc(h, ei3, dinvw.reshape(_NPAD, 64))
    g2 = _mid_tc(ap1, h, dinvc, b1.reshape(1, 64), W2)
    ap2 = _conv2_sc(g2, ei3)
    out = _pool_tc(ap2, g2, dinvc, b2.reshape(1, 64), bat3)
    return out


# speedup vs baseline: 1.2274x; 1.0675x over previous
import functools

import jax
import jax.numpy as jnp
from jax import lax
from jax.experimental import pallas as pl
from jax.experimental.pallas import tpu as pltpu
from jax.experimental.pallas import tpu_sc as plsc

_N = 10000
_E = 320000
_G = 64

_NC = 2
_NS = 16
_NW = _NC * _NS

_EPW = _E // _NW
_EPS = _E // _NS
_CH = 128
_NFULL = _EPW // _CH
_TAIL = _EPW - _NFULL * _CH

_NPAD = 10240
_RPT = _NPAD // _NS
_SPT = _N // _NS
_BLK = 2000
_GRID = _N // _BLK


def _wid(cid, sid):
    return cid * _NS + sid


def _deg_body(ei16, out, outw, dst_v, ones_v, wbuf_v, wide_v, deg_sh):
    cid = lax.axis_index("c")
    sid = lax.axis_index("s")

    ones16 = jnp.full((16,), 1.0, jnp.float32)

    def ob(i, _):
        ones_v[pl.ds(i * 16, 16)] = ones16
        return ()
    lax.fori_loop(0, _EPW // 16, ob, ())
    zeros16 = jnp.zeros((16,), jnp.float32)

    def zb(i, _):
        wbuf_v[pl.ds(i * 16, 16)] = zeros16
        return ()
    lax.fori_loop(0, _RPT // 16, zb, ())
    pltpu.sync_copy(wbuf_v, deg_sh.at[pl.ds(sid * _RPT, _RPT)])
    plsc.subcore_barrier()

    pltpu.sync_copy(ei16.at[1, sid], dst_v)
    pltpu.sync_copy(ones_v, deg_sh.at[dst_v.at[pl.ds(0, _EPW)]], add=True)
    pltpu.sync_copy(ones_v, deg_sh.at[dst_v.at[pl.ds(_EPW, _EPW)]], add=True)
    plsc.subcore_barrier()

    pltpu.sync_copy(deg_sh.at[pl.ds(sid * _RPT, _RPT)], wbuf_v)
    magic = jnp.full((16,), 0x5F3759DF, jnp.int32)

    def nw(i, _):
        x = wbuf_v[pl.ds(i * 16, 16)] + 1.0
        y = plsc.bitcast(magic - (plsc.bitcast(x, jnp.int32) >> 1),
                         jnp.float32)
        for _u in range(3):
            y = y * (1.5 - 0.5 * x * y * y)
        wbuf_v[pl.ds(i * 16, 16)] = y
        for k in range(16):
            r = i * 16 + k
            dvv = jnp.broadcast_to(y[k], (16,))
            dvb = plsc.pack(dvv, dvv, format=plsc.PackFormat.INTERLEAVED)
            for j in range(2):
                wide_v[pl.ds(r * 64 + j * 32, 32)] = dvb
        return ()
    lax.fori_loop(0, _RPT // 16, nw, ())

    @pl.when(cid == 0)
    def _():
        pltpu.sync_copy(wbuf_v, out.at[pl.ds(sid * _RPT, _RPT)])
        pltpu.sync_copy(wide_v, outw.at[pl.ds(sid * _RPT * 64, _RPT * 64)])


def _deg_sc(ei16):
    mesh = plsc.VectorSubcoreMesh(core_axis_name="c", subcore_axis_name="s")
    return pl.kernel(
        _deg_body,
        out_type=(jax.ShapeDtypeStruct((_NPAD,), jnp.float32),
                  jax.ShapeDtypeStruct((_NPAD * 64,), jnp.bfloat16)),
        mesh=mesh,
        compiler_params=pltpu.CompilerParams(use_tc_tiling_on_sc=False,
                                             needs_layout_passes=False),
        scratch_types=[
            pltpu.VMEM((_EPS,), jnp.int32),
            pltpu.VMEM((_EPW,), jnp.float32),
            pltpu.VMEM((_RPT,), jnp.float32),
            pltpu.VMEM((_RPT * 64,), jnp.bfloat16),
            pltpu.VMEM_SHARED((_NPAD,), jnp.float32),
        ],
    )(ei16)


_NBUF = 3


def _edge_pass(g_sh, acc_sh, src_v, dst_v, bufs, gsem, ssem):
    def sg(j, b):
        return pltpu.async_copy(g_sh.at[src_v.at[pl.ds(j * _CH, _CH)]],
                                bufs[b], gsem[b])

    def ss(j, b):
        return pltpu.async_copy(bufs[b],
                                acc_sh.at[dst_v.at[pl.ds(j * _CH, _CH)]],
                                ssem[b], add=True)

    gh = [None] * _NBUF
    sh = [None] * _NBUF
    gh[0] = sg(0, 0)
    for j in range(_NFULL):
        b = j % _NBUF
        if j + 1 < _NFULL:
            bb = (j + 1) % _NBUF
            if sh[bb] is not None:
                sh[bb].wait()
            gh[bb] = sg(j + 1, bb)
        gh[b].wait()
        sh[b] = ss(j, b)
    for k in range(min(_NBUF, _NFULL)):
        b = (_NFULL - 1 - k) % _NBUF
        if sh[b] is not None:
            sh[b].wait()
            sh[b] = None
    pltpu.sync_copy(g_sh.at[src_v.at[pl.ds(_NFULL * _CH, _TAIL)]],
                    bufs[0].at[pl.ds(0, _TAIL)])
    pltpu.sync_copy(bufs[0].at[pl.ds(0, _TAIL)],
                    acc_sh.at[dst_v.at[pl.ds(_NFULL * _CH, _TAIL)]],
                    add=True)


def _zero_acc(zbuf_v, acc_sh, sid):
    zeros32 = jnp.zeros((32,), jnp.bfloat16)

    def zb(i, _):
        for j in range(2):
            zbuf_v[i, pl.ds(j * 32, 32)] = zeros32
        return ()
    lax.fori_loop(0, 64, zb, ())
    for r in range(_RPT // 64):
        pltpu.sync_copy(zbuf_v, acc_sh.at[pl.ds(sid * _RPT + r * 64, 64)])


def _conv1_body(h, ei3, dinvw, out, src_v, dst_v, b0, b1, b2, zbuf_v,
                g0, g1, g2, s0, s1, s2, hbuf_v, dwbuf_v, acc_sh, g_sh):
    bufs, gsem, ssem = (b0, b1, b2), (g0, g1, g2), (s0, s1, s2)
    cid = lax.axis_index("c")
    sid = lax.axis_index("s")

    last = _NS - 1
    ltr = _N - last * _RPT

    @pl.when(sid < last)
    def _():
        pltpu.sync_copy(h.at[pl.ds(sid * _RPT, _RPT)], hbuf_v)
        pltpu.sync_copy(dinvw.at[pl.ds(sid * _RPT, _RPT)], dwbuf_v)

    @pl.when(sid == last)
    def _():
        pltpu.sync_copy(h.at[pl.ds(last * _RPT, ltr)],
                        hbuf_v.at[pl.ds(0, ltr)])
        pltpu.sync_copy(dinvw.at[pl.ds(last * _RPT, ltr)],
                        dwbuf_v.at[pl.ds(0, ltr)])
    _zero_acc(zbuf_v, acc_sh, sid)

    def sc_row(r, _):
        for j in range(2):
            hbuf_v[r, pl.ds(j * 32, 32)] = \
                hbuf_v[r, pl.ds(j * 32, 32)] * dwbuf_v[r, pl.ds(j * 32, 32)]
        return ()
    nrow = jnp.where(sid == last, ltr, _RPT)
    lax.fori_loop(0, nrow, sc_row, ())

    @pl.when(sid < last)
    def _():
        pltpu.sync_copy(hbuf_v, g_sh.at[pl.ds(sid * _RPT, _RPT)])

    @pl.when(sid == last)
    def _():
        pltpu.sync_copy(hbuf_v.at[pl.ds(0, ltr)],
                        g_sh.at[pl.ds(last * _RPT, ltr)])
    plsc.subcore_barrier()

    w = _wid(cid, sid)
    pltpu.sync_copy(ei3.at[0, w], src_v)
    pltpu.sync_copy(ei3.at[1, w], dst_v)
    _edge_pass(g_sh, acc_sh, src_v, dst_v, bufs, gsem, ssem)
    plsc.subcore_barrier()

    pltpu.sync_copy(acc_sh.at[pl.ds(sid * _RPT, _RPT)],
                    out.at[cid, pl.ds(sid * _RPT, _RPT)])


def _conv2_body(g, ei3, out, src_v, dst_v, b0, b1, b2, zbuf_v,
                g0, g1, g2, s0, s1, s2, acc_sh, g_sh):
    bufs, gsem, ssem = (b0, b1, b2), (g0, g1, g2), (s0, s1, s2)
    cid = lax.axis_index("c")
    sid = lax.axis_index("s")

    pltpu.sync_copy(g.at[pl.ds(sid * _SPT, _SPT)],
                    g_sh.at[pl.ds(sid * _SPT, _SPT)])
    _zero_acc(zbuf_v, acc_sh, sid)
    plsc.subcore_barrier()

    w = _wid(cid, sid)
    pltpu.sync_copy(ei3.at[0, w], src_v)
    pltpu.sync_copy(ei3.at[1, w], dst_v)
    _edge_pass(g_sh, acc_sh, src_v, dst_v, bufs, gsem, ssem)
    plsc.subcore_barrier()

    pltpu.sync_copy(acc_sh.at[pl.ds(sid * _RPT, _RPT)],
                    out.at[cid, pl.ds(sid * _RPT, _RPT)])


_CONV_OUT = jax.ShapeDtypeStruct((_NC, _NPAD, 64), jnp.bfloat16)
_CONV_SCRATCH = (
    [pltpu.VMEM((_EPW,), jnp.int32),
     pltpu.VMEM((_EPW,), jnp.int32)]
    + [pltpu.VMEM((_CH, 64), jnp.bfloat16)] * _NBUF
    + [pltpu.VMEM((64, 64), jnp.bfloat16)]
    + [pltpu.SemaphoreType.DMA] * (2 * _NBUF)
)
_CONV_SHARED = [
    pltpu.VMEM_SHARED((_NPAD, 64), jnp.bfloat16),
    pltpu.VMEM_SHARED((_NPAD, 64), jnp.bfloat16),
]


def _conv1_sc(h, ei3, dinvw):
    mesh = plsc.VectorSubcoreMesh(core_axis_name="c", subcore_axis_name="s")
    return pl.kernel(
        _conv1_body,
        out_type=_CONV_OUT,
        mesh=mesh,
        compiler_params=pltpu.CompilerParams(use_tc_tiling_on_sc=False),
        scratch_types=_CONV_SCRATCH + [
            pltpu.VMEM((_RPT, 64), jnp.bfloat16),
            pltpu.VMEM((_RPT, 64), jnp.bfloat16),
        ] + _CONV_SHARED,
    )(h, ei3, dinvw)


def _conv2_sc(g, ei3):
    mesh = plsc.VectorSubcoreMesh(core_axis_name="c", subcore_axis_name="s")
    return pl.kernel(
        _conv2_body,
        out_type=_CONV_OUT,
        mesh=mesh,
        compiler_params=pltpu.CompilerParams(use_tc_tiling_on_sc=False),
        scratch_types=_CONV_SCRATCH + _CONV_SHARED,
    )(g, ei3)


def _mm1_body(x_ref, w_ref, h_ref):
    h_ref[...] = jnp.dot(x_ref[...], w_ref[...],
                         preferred_element_type=jnp.float32
                         ).astype(jnp.bfloat16)


def _mid_body(ap_ref, h_ref, dv_ref, b1_ref, w2_ref, g2_ref):
    dinv = dv_ref[...]
    g1 = dinv * h_ref[...].astype(jnp.float32)
    acc = (ap_ref[0] + ap_ref[1]).astype(jnp.float32) + g1
    h1 = jnp.maximum(dinv * acc + b1_ref[...], 0.0)
    g2_ref[...] = (dinv * jnp.dot(h1, w2_ref[...],
                                  preferred_element_type=jnp.float32)
                   ).astype(jnp.bfloat16)


def _pool_body(ap_ref, g2_ref, dv_ref, b2_ref, bat_ref, out_ref):
    i = pl.program_id(0)
    dinv = dv_ref[...]
    h2 = dinv * ((ap_ref[0] + ap_ref[1]).astype(jnp.float32)
                 + g2_ref[...].astype(jnp.float32)) + b2_ref[...]
    ids = jax.lax.broadcasted_iota(jnp.int32, (_G, _BLK), 0)
    oht = (ids == bat_ref[0]).astype(jnp.float32)
    part = jnp.dot(oht, h2, preferred_element_type=jnp.float32)

    @pl.when(i == 0)
    def _():
        out_ref[...] = part

    @pl.when(i > 0)
    def _():
        out_ref[...] += part


def _mm1_tc(x, W1):
    return pl.pallas_call(
        _mm1_body,
        grid=(_GRID,),
        in_specs=[pl.BlockSpec((_BLK, 128), lambda i: (i, 0)),
                  pl.BlockSpec((128, 64), lambda i: (0, 0))],
        out_specs=pl.BlockSpec((_BLK, 64), lambda i: (i, 0)),
        out_shape=jax.ShapeDtypeStruct((_N, 64), jnp.bfloat16),
    )(x, W1)


def _mid_tc(ap, h, dinvc, b1, W2):
    return pl.pallas_call(
        _mid_body,
        grid=(_GRID,),
        in_specs=[pl.BlockSpec((_NC, _BLK, 64), lambda i: (0, i, 0)),
                  pl.BlockSpec((_BLK, 64), lambda i: (i, 0)),
                  pl.BlockSpec((_BLK, 1), lambda i: (i, 0)),
                  pl.BlockSpec((1, 64), lambda i: (0, 0)),
                  pl.BlockSpec((64, 64), lambda i: (0, 0))],
        out_specs=pl.BlockSpec((_BLK, 64), lambda i: (i, 0)),
        out_shape=jax.ShapeDtypeStruct((_N, 64), jnp.bfloat16),
    )(ap, h, dinvc, b1, W2)


def _pool_tc(ap, g2, dinvc, b2, bat3):
    return pl.pallas_call(
        _pool_body,
        grid=(_GRID,),
        in_specs=[pl.BlockSpec((_NC, _BLK, 64), lambda i: (0, i, 0)),
                  pl.BlockSpec((_BLK, 64), lambda i: (i, 0)),
                  pl.BlockSpec((_BLK, 1), lambda i: (i, 0)),
                  pl.BlockSpec((1, 64), lambda i: (0, 0)),
                  pl.BlockSpec((1, 1, _BLK), lambda i: (i, 0, 0))],
        out_specs=pl.BlockSpec((_G, 64), lambda i: (0, 0)),
        out_shape=jax.ShapeDtypeStruct((_G, 64), jnp.float32),
    )(ap, g2, dinvc, b2, bat3)


def kernel(x, edge_index, batch, W1, b1, W2, b2):
    ei3 = edge_index.reshape(2, _NW, _EPW)
    ei16 = edge_index.reshape(2, _NS, _EPS)
    bat3 = batch.reshape(_GRID, 1, _BLK)

    dinv, dinvw = _deg_sc(ei16)
    h = _mm1_tc(x, W1)
    dinvc = dinv.reshape(_NPAD, 1)

    ap1 = _conv1_sc(h, ei3, dinvw.reshape(_NPAD, 64))
    g2 = _mid_tc(ap1, h, dinvc, b1.reshape(1, 64), W2)
    ap2 = _conv2_sc(g2, ei3)
    out = _pool_tc(ap2, g2, dinvc, b2.reshape(1, 64), bat3)
    return out
